# baseline (speedup 1.0000x reference)
import jax
import jax.numpy as jnp
from jax import lax
from jax.experimental import pallas as pl
from jax.experimental.pallas import tpu as pltpu
from jax.experimental.pallas import tpu_sc as plsc

N_NODES = 10000
N_EDGES = 320000
D = 128

NC = 2
NS = 16
NW = NC * NS

PB = 128
BLKS_W = 80
E_PAD = NW * BLKS_W * PB

ACC_ROWS = 10240
DEG_ROWS = 10240
DROWS_S = DEG_ROWS // NS


def _mesh():
    return plsc.VectorSubcoreMesh(core_axis_name="c", subcore_axis_name="s")


def _deg_body(colp3, deg_out, dacc2, colv, degl, part, outv):
    c = lax.axis_index("c")
    s = lax.axis_index("s")
    w = c * NS + s
    pltpu.sync_copy(colp3.at[pl.ds(w * BLKS_W, BLKS_W)], colv)

    def zstep(i, _):
        degl[pl.ds(i * 16, 16)] = jnp.zeros((16,), jnp.float32)
        return _

    lax.fori_loop(0, DEG_ROWS // 16, zstep, None)

    ones16 = jnp.ones((16,), jnp.float32)

    def step(j, _):
        for k in range(PB // 16):
            idx = colv[j, pl.ds(k * 16, 16)]
            plsc.addupdate_scatter(degl, [idx], ones16)
        return _

    lax.fori_loop(0, BLKS_W, step, None)
    pltpu.sync_copy(degl, dacc2.at[s])
    plsc.subcore_barrier()
    for k in range(NS):
        pltpu.sync_copy(dacc2.at[k, pl.ds(s * DROWS_S, DROWS_S)], part.at[k])

    def rstep(i, _):
        a = part[0, pl.ds(i * 16, 16)]
        for k in range(1, NS):
            a = a + part[k, pl.ds(i * 16, 16)]
        outv[pl.ds(i * 16, 16)] = a
        return _

    lax.fori_loop(0, DROWS_S // 16, rstep, None)
    pltpu.sync_copy(outv, deg_out.at[c, pl.ds(s * DROWS_S, DROWS_S)])


def _deg_kernel(colp3):
    f = pl.kernel(
        _deg_body,
        out_type=jax.ShapeDtypeStruct((NC, DEG_ROWS), jnp.float32),
        mesh=_mesh(),
        compiler_params=pltpu.CompilerParams(needs_layout_passes=False),
        scratch_types=[
            pltpu.VMEM_SHARED((NS, DEG_ROWS), jnp.float32),
            pltpu.VMEM((BLKS_W, PB), jnp.int32),
            pltpu.VMEM((DEG_ROWS,), jnp.float32),
            pltpu.VMEM((NS, DROWS_S), jnp.float32),
            pltpu.VMEM((DROWS_S,), jnp.float32),
        ],
    )
    return f(colp3)


NBUF = 2
IC = 40


def _prop_body(g, rowp3, colp3, zerosb, acc2, acc, rowv, colv, bufs,
               gs0, gs1, ssem):
    c = lax.axis_index("c")
    s = lax.axis_index("s")
    w = c * NS + s
    gsems = (gs0, gs1)
    pltpu.sync_copy(zerosb, acc.at[pl.ds(s * (ACC_ROWS // NS), ACC_ROWS // NS)])
    plsc.subcore_barrier()

    def chunk(i, _):
        base = w * BLKS_W + i * IC
        pltpu.sync_copy(rowp3.at[pl.ds(base, IC)], rowv)
        pltpu.sync_copy(colp3.at[pl.ds(base, IC)], colv)
        for b in range(NBUF):
            pltpu.async_copy(g.at[rowv.at[b]], bufs.at[b], gsems[b])
        for j in range(IC):
            b = j % NBUF
            pltpu.make_async_copy(g.at[rowv.at[j]], bufs.at[b], gsems[b]).wait()
            pltpu.async_copy(bufs.at[b], acc.at[colv.at[j]], ssem, add=True).wait()
            if j + NBUF < IC:
                pltpu.async_copy(g.at[rowv.at[j + NBUF]], bufs.at[b], gsems[b])
        return _

    lax.fori_loop(0, BLKS_W // IC, chunk, None)
    plsc.subcore_barrier()
    rows = ACC_ROWS // NS
    pltpu.sync_copy(acc.at[pl.ds(s * rows, rows)], acc2.at[c, pl.ds(s * rows, rows)])


def _prop_kernel(g, rowp3, colp3, zerosb):
    f = pl.kernel(
        _prop_body,
        out_type=jax.ShapeDtypeStruct((NC, ACC_ROWS, D), jnp.float32),
        mesh=_mesh(),
        scratch_types=[
            pltpu.VMEM_SHARED((ACC_ROWS, D), jnp.float32),
            pltpu.VMEM((IC, PB), jnp.int32),
            pltpu.VMEM((IC, PB), jnp.int32),
            pltpu.VMEM((NBUF, PB, D), jnp.float32),
            pltpu.SemaphoreType.DMA,
            pltpu.SemaphoreType.DMA,
            pltpu.SemaphoreType.DMA,
        ],
    )
    return f(g, rowp3, colp3, zerosb)


def _mm_body(x_ref, w_ref, d0_ref, d1_ref, g_ref, deg_ref):
    h = jnp.dot(x_ref[...], w_ref[...], preferred_element_type=jnp.float32)
    d = d0_ref[...] + d1_ref[...]
    deg_ref[...] = d
    dis = jnp.where(d > 0, lax.rsqrt(d), 0.0)
    g_ref[...] = h * dis


def _mm_kernel(x, W, d0, d1):
    bm = 400
    grid = N_NODES // bm
    return pl.pallas_call(
        _mm_body,
        grid=(grid,),
        in_specs=[
            pl.BlockSpec((bm, D), lambda i: (i, 0)),
            pl.BlockSpec((D, D), lambda i: (0, 0)),
            pl.BlockSpec((bm, 1), lambda i: (i, 0)),
            pl.BlockSpec((bm, 1), lambda i: (i, 0)),
        ],
        out_specs=[
            pl.BlockSpec((bm, D), lambda i: (i, 0)),
            pl.BlockSpec((bm, 1), lambda i: (i, 0)),
        ],
        out_shape=[
            jax.ShapeDtypeStruct((N_NODES, D), jnp.float32),
            jax.ShapeDtypeStruct((N_NODES, 1), jnp.float32),
        ],
    )(x, W, d0, d1)


def _fin_body(a_ref, d_ref, b_ref, o_ref):
    a = a_ref[0] + a_ref[1]
    d = d_ref[...]
    dis = jnp.where(d > 0, lax.rsqrt(d), 0.0)
    o_ref[...] = jax.nn.sigmoid(a * dis + b_ref[...])


def _fin_kernel(acc2, deg, b):
    bm = 400
    grid = N_NODES // bm
    return pl.pallas_call(
        _fin_body,
        grid=(grid,),
        in_specs=[
            pl.BlockSpec((NC, bm, D), lambda i: (0, i, 0)),
            pl.BlockSpec((bm, 1), lambda i: (i, 0)),
            pl.BlockSpec((1, D), lambda i: (0, 0)),
        ],
        out_specs=pl.BlockSpec((bm, D), lambda i: (i, 0)),
        out_shape=jax.ShapeDtypeStruct((N_NODES, D), jnp.float32),
    )(acc2, deg, b)


def kernel(x, edge_index, W, b):
    row = edge_index[0].astype(jnp.int32)
    col = edge_index[1].astype(jnp.int32)
    pad = E_PAD - N_EDGES
    pad_src = jnp.arange(pad, dtype=jnp.int32) % N_NODES
    pad_dst = N_NODES + (jnp.arange(pad, dtype=jnp.int32) % (ACC_ROWS - N_NODES))
    rowp = jnp.concatenate([row, pad_src])
    colp = jnp.concatenate([col, pad_dst])
    rowp3 = rowp.reshape(-1, PB)
    colp3 = colp.reshape(-1, PB)

    zerosb = jnp.zeros((ACC_ROWS // NS, D), jnp.float32)

    degv = _deg_kernel(colp3)
    d0 = degv[0, :N_NODES].reshape(N_NODES, 1)
    d1 = degv[1, :N_NODES].reshape(N_NODES, 1)
    g, deg = _mm_kernel(x, W, d0, d1)
    acc2 = _prop_kernel(g, rowp3, colp3, zerosb)
    out = _fin_kernel(acc2, deg, b.reshape(1, D))
    return out


# speedup vs baseline: 1.0094x; 1.0094x over previous
import jax
import jax.numpy as jnp
from jax import lax
from jax.experimental import pallas as pl
from jax.experimental.pallas import tpu as pltpu
from jax.experimental.pallas import tpu_sc as plsc

N_NODES = 10000
N_EDGES = 320000
D = 128

NC = 2
NS = 16
NW = NC * NS

PB = 128
BLKS_W = 80
E_PAD = NW * BLKS_W * PB

ACC_ROWS = 10240
DEG_ROWS = 10240
DROWS_S = DEG_ROWS // NS


def _mesh():
    return plsc.VectorSubcoreMesh(core_axis_name="c", subcore_axis_name="s")


def _deg_body(colp3, deg_out, dacc2, colv, degl, part, outv):
    c = lax.axis_index("c")
    s = lax.axis_index("s")
    w = c * NS + s
    pltpu.sync_copy(colp3.at[pl.ds(w * BLKS_W, BLKS_W)], colv)

    def zstep(i, _):
        degl[pl.ds(i * 16, 16)] = jnp.zeros((16,), jnp.float32)
        return _

    lax.fori_loop(0, DEG_ROWS // 16, zstep, None)

    ones16 = jnp.ones((16,), jnp.float32)

    def step(j, _):
        for k in range(PB // 16):
            idx = colv[j, pl.ds(k * 16, 16)]
            plsc.addupdate_scatter(degl, [idx], ones16)
        return _

    lax.fori_loop(0, BLKS_W, step, None)
    pltpu.sync_copy(degl, dacc2.at[s])
    plsc.subcore_barrier()
    for k in range(NS):
        pltpu.sync_copy(dacc2.at[k, pl.ds(s * DROWS_S, DROWS_S)], part.at[k])

    def rstep(i, _):
        a = part[0, pl.ds(i * 16, 16)]
        for k in range(1, NS):
            a = a + part[k, pl.ds(i * 16, 16)]
        outv[pl.ds(i * 16, 16)] = a
        return _

    lax.fori_loop(0, DROWS_S // 16, rstep, None)
    pltpu.sync_copy(outv, deg_out.at[c, pl.ds(s * DROWS_S, DROWS_S)])


def _deg_kernel(colp3):
    f = pl.kernel(
        _deg_body,
        out_type=jax.ShapeDtypeStruct((NC, DEG_ROWS), jnp.float32),
        mesh=_mesh(),
        compiler_params=pltpu.CompilerParams(needs_layout_passes=False),
        scratch_types=[
            pltpu.VMEM_SHARED((NS, DEG_ROWS), jnp.float32),
            pltpu.VMEM((BLKS_W, PB), jnp.int32),
            pltpu.VMEM((DEG_ROWS,), jnp.float32),
            pltpu.VMEM((NS, DROWS_S), jnp.float32),
            pltpu.VMEM((DROWS_S,), jnp.float32),
        ],
    )
    return f(colp3)


NBUF = 2
IC = 40


def _prop_body(g, rowp3, colp3, zerosb, acc2, acc, rowv, colv, bufs,
               gs0, gs1, ssem, isem):
    c = lax.axis_index("c")
    s = lax.axis_index("s")
    w = c * NS + s
    gsems = (gs0, gs1)
    pltpu.async_copy(rowp3.at[pl.ds(w * BLKS_W, IC)], rowv, isem)
    pltpu.async_copy(colp3.at[pl.ds(w * BLKS_W, IC)], colv, isem)
    pltpu.sync_copy(zerosb, acc.at[pl.ds(s * (ACC_ROWS // NS), ACC_ROWS // NS)])
    plsc.subcore_barrier()

    def chunk(i, _):
        base = w * BLKS_W + i * IC

        @pl.when(i == 0)
        def _():
            pltpu.make_async_copy(rowp3.at[pl.ds(base, IC)], rowv, isem).wait()
            pltpu.make_async_copy(colp3.at[pl.ds(base, IC)], colv, isem).wait()

        @pl.when(i > 0)
        def _():
            pltpu.sync_copy(rowp3.at[pl.ds(base, IC)], rowv)
            pltpu.sync_copy(colp3.at[pl.ds(base, IC)], colv)
        for b in range(NBUF):
            pltpu.async_copy(g.at[rowv.at[b]], bufs.at[b], gsems[b])
        for j in range(IC):
            b = j % NBUF
            pltpu.make_async_copy(g.at[rowv.at[j]], bufs.at[b], gsems[b]).wait()
            pltpu.async_copy(bufs.at[b], acc.at[colv.at[j]], ssem, add=True).wait()
            if j + NBUF < IC:
                pltpu.async_copy(g.at[rowv.at[j + NBUF]], bufs.at[b], gsems[b])
        return _

    lax.fori_loop(0, BLKS_W // IC, chunk, None)
    plsc.subcore_barrier()
    rows = ACC_ROWS // NS
    pltpu.sync_copy(acc.at[pl.ds(s * rows, rows)], acc2.at[c, pl.ds(s * rows, rows)])


def _prop_kernel(g, rowp3, colp3, zerosb):
    f = pl.kernel(
        _prop_body,
        out_type=jax.ShapeDtypeStruct((NC, ACC_ROWS, D), jnp.float32),
        mesh=_mesh(),
        scratch_types=[
            pltpu.VMEM_SHARED((ACC_ROWS, D), jnp.float32),
            pltpu.VMEM((IC, PB), jnp.int32),
            pltpu.VMEM((IC, PB), jnp.int32),
            pltpu.VMEM((NBUF, PB, D), jnp.float32),
            pltpu.SemaphoreType.DMA,
            pltpu.SemaphoreType.DMA,
            pltpu.SemaphoreType.DMA,
            pltpu.SemaphoreType.DMA,
        ],
    )
    return f(g, rowp3, colp3, zerosb)


def _mm_body(x_ref, w_ref, d0_ref, d1_ref, g_ref, deg_ref):
    h = jnp.dot(x_ref[...], w_ref[...], preferred_element_type=jnp.float32)
    d = d0_ref[...] + d1_ref[...]
    deg_ref[...] = d
    dis = jnp.where(d > 0, lax.rsqrt(d), 0.0)
    g_ref[...] = h * dis


def _mm_kernel(x, W, d0, d1):
    bm = 400
    grid = N_NODES // bm
    return pl.pallas_call(
        _mm_body,
        grid=(grid,),
        in_specs=[
            pl.BlockSpec((bm, D), lambda i: (i, 0)),
            pl.BlockSpec((D, D), lambda i: (0, 0)),
            pl.BlockSpec((bm, 1), lambda i: (i, 0)),
            pl.BlockSpec((bm, 1), lambda i: (i, 0)),
        ],
        out_specs=[
            pl.BlockSpec((bm, D), lambda i: (i, 0)),
            pl.BlockSpec((bm, 1), lambda i: (i, 0)),
        ],
        out_shape=[
            jax.ShapeDtypeStruct((N_NODES, D), jnp.float32),
            jax.ShapeDtypeStruct((N_NODES, 1), jnp.float32),
        ],
    )(x, W, d0, d1)


def _fin_body(a_ref, d_ref, b_ref, o_ref):
    a = a_ref[0] + a_ref[1]
    d = d_ref[...]
    dis = jnp.where(d > 0, lax.rsqrt(d), 0.0)
    o_ref[...] = jax.nn.sigmoid(a * dis + b_ref[...])


def _fin_kernel(acc2, deg, b):
    bm = 400
    grid = N_NODES // bm
    return pl.pallas_call(
        _fin_body,
        grid=(grid,),
        in_specs=[
            pl.BlockSpec((NC, bm, D), lambda i: (0, i, 0)),
            pl.BlockSpec((bm, 1), lambda i: (i, 0)),
            pl.BlockSpec((1, D), lambda i: (0, 0)),
        ],
        out_specs=pl.BlockSpec((bm, D), lambda i: (i, 0)),
        out_shape=jax.ShapeDtypeStruct((N_NODES, D), jnp.float32),
    )(acc2, deg, b)


def kernel(x, edge_index, W, b):
    row = edge_index[0].astype(jnp.int32)
    col = edge_index[1].astype(jnp.int32)
    pad = E_PAD - N_EDGES
    pad_src = jnp.arange(pad, dtype=jnp.int32) % N_NODES
    pad_dst = N_NODES + (jnp.arange(pad, dtype=jnp.int32) % (ACC_ROWS - N_NODES))
    rowp = jnp.concatenate([row, pad_src])
    colp = jnp.concatenate([col, pad_dst])
    rowp3 = rowp.reshape(-1, PB)
    colp3 = colp.reshape(-1, PB)

    zerosb = jnp.zeros((ACC_ROWS // NS, D), jnp.float32)

    degv = _deg_kernel(colp3)
    d0 = degv[0, :N_NODES].reshape(N_NODES, 1)
    d1 = degv[1, :N_NODES].reshape(N_NODES, 1)
    g, deg = _mm_kernel(x, W, d0, d1)
    acc2 = _prop_kernel(g, rowp3, colp3, zerosb)
    out = _fin_kernel(acc2, deg, b.reshape(1, D))
    return out
